# baseline (speedup 1.0000x reference)
import functools

import jax
import jax.numpy as jnp
from jax import lax
from jax.experimental import pallas as pl
from jax.experimental.pallas import tpu as pltpu
from jax.experimental.pallas import tpu_sc as plsc


def _c_table_body(T, E, K, S, TOK_W, ens_hbm, w_hbm, c_hbm,
                  ens_v, w_v, c_v):
    NC = 2
    wid = lax.axis_index("s") * NC + lax.axis_index("c")
    base = wid * TOK_W
    b = base // S
    s0 = base - b * S
    for k in range(K):
        off = (b * K + k) * S + s0
        pltpu.sync_copy(ens_hbm.at[pl.ds(off, TOK_W)],
                        ens_v.at[pl.ds(k * TOK_W, TOK_W)])
        pltpu.sync_copy(w_hbm.at[pl.ds(off, TOK_W)],
                        w_v.at[pl.ds(k * TOK_W, TOK_W)])
    for ef in range(E):
        col = jnp.zeros((TOK_W,), jnp.float32)
        for k in range(K):
            e = ens_v[pl.ds(k * TOK_W, TOK_W)]
            w = w_v[pl.ds(k * TOK_W, TOK_W)]
            col = col + jnp.where(e == ef, w, 0.0)
        c_v[pl.ds(ef * TOK_W, TOK_W)] = col
    pltpu.sync_copy(c_v, c_hbm.at[pl.ds(wid * E * TOK_W, E * TOK_W)])


def _routing_table_sc(ens_flat, w_flat, T, E, K, S):
    info = plsc.get_sparse_core_info()
    NW = info.num_cores * info.num_subcores
    TOK_W = T // NW
    mesh = plsc.VectorSubcoreMesh(core_axis_name="c", subcore_axis_name="s")
    kfn = pl.kernel(
        functools.partial(_c_table_body, T, E, K, S, TOK_W),
        out_type=jax.ShapeDtypeStruct((T * E,), jnp.float32),
        mesh=mesh,
        scratch_types=[
            pltpu.VMEM((K * TOK_W,), jnp.int32),
            pltpu.VMEM((K * TOK_W,), jnp.float32),
            pltpu.VMEM((E * TOK_W,), jnp.float32),
        ],
    )
    return kfn(ens_flat, w_flat), NW, TOK_W


def _ffe_body(c_ref, x_ref, k0_ref, k1_ref, o_ref):
    NW, E, TOK_W = c_ref.shape
    _, D, BKd = k1_ref.shape
    H = E * BKd
    T = NW * TOK_W

    c = jnp.transpose(c_ref[...], (0, 2, 1)).reshape(T, E)

    blk = lax.broadcasted_iota(jnp.int32, (E, H), 1) // BKd
    expand = jnp.where(lax.broadcasted_iota(jnp.int32, (E, H), 0) == blk,
                       1.0, 0.0)
    scale = jax.lax.dot_general(c, expand, (((1,), (0,)), ((), ())),
                                preferred_element_type=jnp.float32)

    h = jax.lax.dot_general(x_ref[...].astype(jnp.bfloat16),
                            k0_ref[...].astype(jnp.bfloat16),
                            (((1,), (1,)), ((), ())),
                            preferred_element_type=jnp.float32)
    h = jnp.maximum(h, 0.0) * scale

    k1t = jnp.transpose(k1_ref[...].astype(jnp.bfloat16),
                        (0, 2, 1)).reshape(H, D)
    o_ref[...] = jax.lax.dot_general(h.astype(jnp.bfloat16), k1t,
                                     (((1,), (0,)), ((), ())),
                                     preferred_element_type=jnp.float32)


def kernel(x, weights, ensembles, kernels_0, kernels_1):
    B, S, D = x.shape
    E, BK, _ = kernels_0.shape
    _, K, _ = weights.shape
    T = B * S

    c_flat, NW, TOK_W = _routing_table_sc(
        ensembles.astype(jnp.int32).reshape(-1),
        weights.reshape(-1), T, E, K, S)
    c3 = c_flat.reshape(NW, E, TOK_W)

    x2 = x.reshape(T, D)
    k0r = kernels_0.reshape(E * BK, D)

    out = pl.pallas_call(
        _ffe_body,
        out_shape=jax.ShapeDtypeStruct((T, D), jnp.float32),
    )(c3, x2, k0r, kernels_1)

    return out.reshape(B, S, D)


# speedup vs baseline: 1.9679x; 1.9679x over previous
import jax
import jax.numpy as jnp
from jax import lax
from jax.experimental import pallas as pl
from jax.experimental.pallas import tpu as pltpu


def _ffe_body(ens_ref, w_ref, x_ref, k0_ref, k1_ref, o_ref):
    BK_rows, S = ens_ref.shape
    E, D, BKd = k1_ref.shape
    H = E * BKd
    T = x_ref.shape[0]
    B = T // S
    K = BK_rows // B

    iota_e = lax.broadcasted_iota(jnp.int32, (E, 1), 0)
    cols = []
    for b in range(B):
        ct = jnp.zeros((E, S), jnp.float32)
        for k in range(K):
            row = b * K + k
            ct = ct + jnp.where(ens_ref[row][None, :] == iota_e,
                                w_ref[row][None, :], 0.0)
        cols.append(ct)
    cT = jnp.concatenate(cols, axis=1)

    blk = lax.broadcasted_iota(jnp.int32, (E, H), 1) // BKd
    expand = jnp.where(lax.broadcasted_iota(jnp.int32, (E, H), 0) == blk,
                       1.0, 0.0)
    scale = jax.lax.dot_general(cT, expand, (((0,), (0,)), ((), ())),
                                preferred_element_type=jnp.float32)

    h = jax.lax.dot_general(x_ref[...].astype(jnp.bfloat16),
                            k0_ref[...].astype(jnp.bfloat16),
                            (((1,), (1,)), ((), ())),
                            preferred_element_type=jnp.float32)
    h = jnp.maximum(h, 0.0) * scale

    k1t = jnp.transpose(k1_ref[...].astype(jnp.bfloat16),
                        (0, 2, 1)).reshape(H, D)
    o_ref[...] = jax.lax.dot_general(h.astype(jnp.bfloat16), k1t,
                                     (((1,), (0,)), ((), ())),
                                     preferred_element_type=jnp.float32)


def kernel(x, weights, ensembles, kernels_0, kernels_1):
    B, S, D = x.shape
    E, BK, _ = kernels_0.shape
    _, K, _ = weights.shape
    T = B * S

    x2 = x.reshape(T, D)
    ens2 = ensembles.astype(jnp.int32).reshape(B * K, S)
    w2 = weights.reshape(B * K, S)
    k0r = kernels_0.reshape(E * BK, D)

    out = pl.pallas_call(
        _ffe_body,
        out_shape=jax.ShapeDtypeStruct((T, D), jnp.float32),
    )(ens2, w2, x2, k0r, kernels_1)

    return out.reshape(B, S, D)
